# baseline (speedup 1.0000x reference)
import functools

import jax
import jax.numpy as jnp
from jax import lax
from jax.experimental import pallas as pl
from jax.experimental.pallas import tpu as pltpu
from jax.experimental.pallas import tpu_sc as plsc

_NUM_WORKERS = 32
_CHUNK = 128


def _sc_gather(emb_table, idx3d, total, emb):
    nch = (total // _CHUNK) // _NUM_WORKERS
    dt = emb_table.dtype
    mesh = plsc.VectorSubcoreMesh(core_axis_name="c", subcore_axis_name="s")

    @functools.partial(
        pl.kernel,
        mesh=mesh,
        out_type=jax.ShapeDtypeStruct((total, emb), dt),
        compiler_params=pltpu.CompilerParams(use_tc_tiling_on_sc=False),
        scratch_types=[
            pltpu.VMEM((nch, _CHUNK), jnp.int32),
            pltpu.VMEM((_CHUNK, emb), dt),
            pltpu.SemaphoreType.DMA,
        ],
    )
    def gather_kernel(table_hbm, idx_hbm, out_hbm, idx_v, rows_v, sem):
        wid = lax.axis_index("s") * 2 + lax.axis_index("c")
        chunk0 = wid * nch
        pltpu.sync_copy(idx_hbm.at[wid], idx_v)

        def body(j, carry):
            pltpu.async_copy(table_hbm.at[idx_v.at[j]], rows_v, sem).wait()
            pltpu.sync_copy(
                rows_v, out_hbm.at[pl.ds((chunk0 + j) * _CHUNK, _CHUNK)]
            )
            return carry

        lax.fori_loop(0, nch, body, 0)

    return gather_kernel(emb_table, idx3d)


def _tc_matmul(x, w, b, bt=512):
    bsz, k = x.shape
    n = w.shape[0]

    def body(x_ref, w_ref, b_ref, o_ref):
        o_ref[...] = (
            lax.dot_general(
                x_ref[...], w_ref[...], (((1,), (1,)), ((), ())),
                preferred_element_type=jnp.float32,
            )
            + b_ref[...]
        )

    return pl.pallas_call(
        body,
        grid=(bsz // bt,),
        in_specs=[
            pl.BlockSpec((bt, k), lambda i: (i, 0)),
            pl.BlockSpec((n, k), lambda i: (0, 0)),
            pl.BlockSpec((1, n), lambda i: (0, 0)),
        ],
        out_specs=pl.BlockSpec((bt, n), lambda i: (i, 0)),
        out_shape=jax.ShapeDtypeStruct((bsz, n), jnp.float32),
    )(x, w, b)


def kernel(input_x, emb_table, W_fc, b_fc):
    bsz, nd = input_x.shape
    vocab, emb = emb_table.shape
    out_dim = W_fc.shape[0]
    total = bsz * nd

    nch = (total // _CHUNK) // _NUM_WORKERS
    idx3d = input_x.reshape(_NUM_WORKERS, nch, _CHUNK)
    gathered = _sc_gather(emb_table.astype(jnp.bfloat16), idx3d, total, emb)
    x = gathered.reshape(bsz, nd * emb)
    return _tc_matmul(x, W_fc.astype(jnp.bfloat16), b_fc.reshape(1, out_dim))


# speedup vs baseline: 1.0939x; 1.0939x over previous
import functools

import jax
import jax.numpy as jnp
from jax import lax
from jax.experimental import pallas as pl
from jax.experimental.pallas import tpu as pltpu
from jax.experimental.pallas import tpu_sc as plsc

_NUM_WORKERS = 32
_CHUNK = 128


def _sc_gather(emb_table, idx3d, total, emb):
    nch = (total // _CHUNK) // _NUM_WORKERS
    rows_per_w = nch * _CHUNK
    dt = emb_table.dtype
    mesh = plsc.VectorSubcoreMesh(core_axis_name="c", subcore_axis_name="s")

    @functools.partial(
        pl.kernel,
        mesh=mesh,
        out_type=jax.ShapeDtypeStruct((total, emb), dt),
        compiler_params=pltpu.CompilerParams(use_tc_tiling_on_sc=False),
        scratch_types=[
            pltpu.VMEM((nch, _CHUNK), jnp.int32),
            pltpu.VMEM((rows_per_w, emb), dt),
            pltpu.SemaphoreType.DMA,
        ],
    )
    def gather_kernel(table_hbm, idx_hbm, out_hbm, idx_v, rows_v, sem):
        wid = lax.axis_index("s") * 2 + lax.axis_index("c")
        pltpu.sync_copy(idx_hbm.at[wid], idx_v)
        descs = [
            pltpu.async_copy(
                table_hbm.at[idx_v.at[j]],
                rows_v.at[pl.ds(j * _CHUNK, _CHUNK)],
                sem,
            )
            for j in range(nch)
        ]
        for d in descs:
            d.wait()
        pltpu.sync_copy(rows_v, out_hbm.at[pl.ds(wid * rows_per_w, rows_per_w)])

    return gather_kernel(emb_table, idx3d)


def _tc_matmul(x, w, b, bt=512):
    bsz, k = x.shape
    n = w.shape[0]

    def body(x_ref, w_ref, b_ref, o_ref):
        o_ref[...] = (
            lax.dot_general(
                x_ref[...], w_ref[...], (((1,), (1,)), ((), ())),
                preferred_element_type=jnp.float32,
            )
            + b_ref[...]
        )

    return pl.pallas_call(
        body,
        grid=(bsz // bt,),
        in_specs=[
            pl.BlockSpec((bt, k), lambda i: (i, 0)),
            pl.BlockSpec((n, k), lambda i: (0, 0)),
            pl.BlockSpec((1, n), lambda i: (0, 0)),
        ],
        out_specs=pl.BlockSpec((bt, n), lambda i: (i, 0)),
        out_shape=jax.ShapeDtypeStruct((bsz, n), jnp.float32),
    )(x, w, b)


def kernel(input_x, emb_table, W_fc, b_fc):
    bsz, nd = input_x.shape
    vocab, emb = emb_table.shape
    out_dim = W_fc.shape[0]
    total = bsz * nd

    nch = (total // _CHUNK) // _NUM_WORKERS
    idx3d = input_x.reshape(_NUM_WORKERS, nch, _CHUNK)
    gathered = _sc_gather(emb_table.astype(jnp.bfloat16), idx3d, total, emb)
    x = gathered.reshape(bsz, nd * emb)
    return _tc_matmul(x, W_fc.astype(jnp.bfloat16), b_fc.reshape(1, out_dim))


# speedup vs baseline: 1.1037x; 1.0089x over previous
import functools

import jax
import jax.numpy as jnp
from jax import lax
from jax.experimental import pallas as pl
from jax.experimental.pallas import tpu as pltpu
from jax.experimental.pallas import tpu_sc as plsc

_NUM_WORKERS = 32
_CHUNK = 128
_RCH = 3


def _sc_gather_packed(table2, idx1d, total):
    pack = table2.shape[1]
    per_w = total // _NUM_WORKERS
    nch = per_w // _CHUNK
    nrounds = (nch + _RCH - 1) // _RCH
    mesh = plsc.VectorSubcoreMesh(core_axis_name="c", subcore_axis_name="s")

    @functools.partial(
        pl.kernel,
        mesh=mesh,
        out_type=jax.ShapeDtypeStruct((total, pack), jnp.float32),
        compiler_params=pltpu.CompilerParams(use_tc_tiling_on_sc=False),
        scratch_types=[
            pltpu.VMEM((per_w,), jnp.int32),
            pltpu.VMEM((_RCH * _CHUNK, pack), jnp.float32),
            pltpu.VMEM((_RCH * _CHUNK, pack), jnp.float32),
            pltpu.SemaphoreType.DMA,
            pltpu.SemaphoreType.DMA,
        ],
    )
    def gather_kernel(table_hbm, idx_hbm, out_hbm, idx_v, buf_a, buf_b, gsem, wsem):
        wid = lax.axis_index("s") * 2 + lax.axis_index("c")
        base = wid * per_w
        pltpu.sync_copy(idx_hbm.at[pl.ds(base, per_w)], idx_v)
        bufs = (buf_a, buf_b)
        wdescs = [None, None]
        for r in range(nrounds):
            buf = bufs[r % 2]
            sz = min(_RCH, nch - r * _RCH)
            if wdescs[r % 2] is not None:
                wdescs[r % 2].wait()
            gds = [
                pltpu.async_copy(
                    table_hbm.at[idx_v.at[pl.ds((r * _RCH + c) * _CHUNK, _CHUNK)]],
                    buf.at[pl.ds(c * _CHUNK, _CHUNK)],
                    gsem,
                )
                for c in range(sz)
            ]
            for d in gds:
                d.wait()
            wdescs[r % 2] = pltpu.async_copy(
                buf.at[pl.ds(0, sz * _CHUNK)],
                out_hbm.at[pl.ds(base + r * _RCH * _CHUNK, sz * _CHUNK)],
                wsem,
            )
        for d in wdescs:
            if d is not None:
                d.wait()

    return gather_kernel(table2, idx1d)


def _tc_select_matmul(x3, par, w3, b, bt=4096):
    nd, bsz, pack = x3.shape
    n = w3.shape[2]

    def body(x_ref, p_ref, w_ref, b_ref, o_ref):
        d = pl.program_id(1)
        xb = x_ref[0]
        pf = p_ref[0]
        sel = jnp.where(pf == 1.0, xb[:, pack // 2 :], xb[:, : pack // 2])
        acc = lax.dot_general(
            sel, w_ref[0], (((1,), (0,)), ((), ())),
            preferred_element_type=jnp.float32,
        )

        @pl.when(d == 0)
        def _():
            o_ref[...] = b_ref[...] + acc

        @pl.when(d != 0)
        def _():
            o_ref[...] += acc

    return pl.pallas_call(
        body,
        grid=(bsz // bt, nd),
        in_specs=[
            pl.BlockSpec((1, bt, pack), lambda i, d: (d, i, 0)),
            pl.BlockSpec((1, bt, 1), lambda i, d: (d, i, 0)),
            pl.BlockSpec((1, pack // 2, n), lambda i, d: (d, 0, 0)),
            pl.BlockSpec((1, n), lambda i, d: (0, 0)),
        ],
        out_specs=pl.BlockSpec((bt, n), lambda i, d: (i, 0)),
        out_shape=jax.ShapeDtypeStruct((bsz, n), jnp.float32),
    )(x3, par, w3, b)


def kernel(input_x, emb_table, W_fc, b_fc):
    bsz, nd = input_x.shape
    vocab, emb = emb_table.shape
    out_dim = W_fc.shape[0]
    total = bsz * nd

    table2 = emb_table.reshape(vocab // 2, 2 * emb)
    idx_t = input_x.T.reshape(-1)
    gathered = _sc_gather_packed(table2, idx_t >> 1, total)
    x3 = gathered.reshape(nd, bsz, 2 * emb)
    par = (idx_t & 1).astype(jnp.float32).reshape(nd, bsz, 1)
    w3 = W_fc.T.reshape(nd, emb, out_dim)
    return _tc_select_matmul(x3, par, w3, b_fc.reshape(1, out_dim))


# speedup vs baseline: 1.1415x; 1.0343x over previous
import functools

import jax
import jax.numpy as jnp
from jax import lax
from jax.experimental import pallas as pl
from jax.experimental.pallas import tpu as pltpu
from jax.experimental.pallas import tpu_sc as plsc

_NUM_WORKERS = 32
_CHUNK = 128
_RCH = 3


def _sc_gather_packed(table2, idx1d, total):
    pack = table2.shape[1]
    per_w = total // _NUM_WORKERS
    nch = per_w // _CHUNK
    nrounds = (nch + _RCH - 1) // _RCH
    mesh = plsc.VectorSubcoreMesh(core_axis_name="c", subcore_axis_name="s")

    @functools.partial(
        pl.kernel,
        mesh=mesh,
        out_type=jax.ShapeDtypeStruct((total, pack), jnp.float32),
        compiler_params=pltpu.CompilerParams(use_tc_tiling_on_sc=False),
        scratch_types=[
            pltpu.VMEM((per_w,), jnp.int32),
            pltpu.VMEM((_RCH * _CHUNK, pack), jnp.float32),
            pltpu.VMEM((_RCH * _CHUNK, pack), jnp.float32),
            pltpu.SemaphoreType.DMA,
            pltpu.SemaphoreType.DMA,
        ],
    )
    def gather_kernel(table_hbm, idx_hbm, out_hbm, idx_v, buf_a, buf_b, gsem, wsem):
        wid = lax.axis_index("s") * 2 + lax.axis_index("c")
        base = wid * per_w
        pltpu.sync_copy(idx_hbm.at[pl.ds(base, per_w)], idx_v)
        bufs = (buf_a, buf_b)
        wdescs = [None, None]
        for r in range(nrounds):
            buf = bufs[r % 2]
            sz = min(_RCH, nch - r * _RCH)
            if wdescs[r % 2] is not None:
                wdescs[r % 2].wait()
            gds = [
                pltpu.async_copy(
                    table_hbm.at[idx_v.at[pl.ds((r * _RCH + c) * _CHUNK, _CHUNK)]],
                    buf.at[pl.ds(c * _CHUNK, _CHUNK)],
                    gsem,
                )
                for c in range(sz)
            ]
            for d in gds:
                d.wait()
            wdescs[r % 2] = pltpu.async_copy(
                buf.at[pl.ds(0, sz * _CHUNK)],
                out_hbm.at[pl.ds(base + r * _RCH * _CHUNK, sz * _CHUNK)],
                wsem,
            )
        for d in wdescs:
            if d is not None:
                d.wait()

    return gather_kernel(table2, idx1d)


def _tc_select_matmul(x3, par, w3, b, bt=4096):
    nd, bsz, pack = x3.shape
    n = w3.shape[2]

    def body(x_ref, p_ref, w_ref, b_ref, o_ref):
        d = pl.program_id(1)
        xb = x_ref[0]
        pf = p_ref[0]
        sel = jnp.where(pf == 1.0, xb[:, pack // 2 :], xb[:, : pack // 2])
        acc = lax.dot_general(
            sel.astype(jnp.bfloat16), w_ref[0], (((1,), (0,)), ((), ())),
            preferred_element_type=jnp.float32,
        )

        @pl.when(d == 0)
        def _():
            o_ref[...] = b_ref[...] + acc

        @pl.when(d != 0)
        def _():
            o_ref[...] += acc

    return pl.pallas_call(
        body,
        grid=(bsz // bt, nd),
        in_specs=[
            pl.BlockSpec((1, bt, pack), lambda i, d: (d, i, 0)),
            pl.BlockSpec((1, bt, 1), lambda i, d: (d, i, 0)),
            pl.BlockSpec((1, pack // 2, n), lambda i, d: (d, 0, 0)),
            pl.BlockSpec((1, n), lambda i, d: (0, 0)),
        ],
        out_specs=pl.BlockSpec((bt, n), lambda i, d: (i, 0)),
        out_shape=jax.ShapeDtypeStruct((bsz, n), jnp.float32),
    )(x3, par, w3, b)


def kernel(input_x, emb_table, W_fc, b_fc):
    bsz, nd = input_x.shape
    vocab, emb = emb_table.shape
    out_dim = W_fc.shape[0]
    total = bsz * nd

    table2 = emb_table.reshape(vocab // 2, 2 * emb)
    idx_t = input_x.T.reshape(-1)
    gathered = _sc_gather_packed(table2, idx_t >> 1, total)
    x3 = gathered.reshape(nd, bsz, 2 * emb)
    par = (idx_t & 1).astype(jnp.float32).reshape(nd, bsz, 1)
    w3 = W_fc.T.reshape(nd, emb, out_dim).astype(jnp.bfloat16)
    return _tc_select_matmul(x3, par, w3, b_fc.reshape(1, out_dim))


# speedup vs baseline: 1.2652x; 1.1084x over previous
import functools

import jax
import jax.numpy as jnp
from jax import lax
from jax.experimental import pallas as pl
from jax.experimental.pallas import tpu as pltpu
from jax.experimental.pallas import tpu_sc as plsc

_NUM_WORKERS = 32
_CHUNK = 128
_RCH = 3


def _sc_gather_packed(table2, idx1d, total):
    pack = table2.shape[1]
    per_w = total // _NUM_WORKERS
    nch = per_w // _CHUNK
    nrounds = (nch + _RCH - 1) // _RCH
    mesh = plsc.VectorSubcoreMesh(core_axis_name="c", subcore_axis_name="s")

    @functools.partial(
        pl.kernel,
        mesh=mesh,
        out_type=jax.ShapeDtypeStruct((total, pack), jnp.float32),
        compiler_params=pltpu.CompilerParams(use_tc_tiling_on_sc=False),
        scratch_types=[
            pltpu.VMEM((per_w,), jnp.int32),
            pltpu.VMEM((_RCH * _CHUNK, pack), jnp.float32),
            pltpu.VMEM((_RCH * _CHUNK, pack), jnp.float32),
            pltpu.SemaphoreType.DMA,
            pltpu.SemaphoreType.DMA,
        ],
    )
    def gather_kernel(table_hbm, idx_hbm, out_hbm, idx_v, buf_a, buf_b, gsem, wsem):
        wid = lax.axis_index("s") * 2 + lax.axis_index("c")
        base = wid * per_w
        pltpu.sync_copy(idx_hbm.at[pl.ds(base, per_w)], idx_v)
        bufs = (buf_a, buf_b)
        wdescs = [None, None]
        for r in range(nrounds):
            buf = bufs[r % 2]
            sz = min(_RCH, nch - r * _RCH)
            if wdescs[r % 2] is not None:
                wdescs[r % 2].wait()
            gds = [
                pltpu.async_copy(
                    table_hbm.at[idx_v.at[pl.ds((r * _RCH + c) * _CHUNK, _CHUNK)]],
                    buf.at[pl.ds(c * _CHUNK, _CHUNK)],
                    gsem,
                )
                for c in range(sz)
            ]
            for d in gds:
                d.wait()
            wdescs[r % 2] = pltpu.async_copy(
                buf.at[pl.ds(0, sz * _CHUNK)],
                out_hbm.at[pl.ds(base + r * _RCH * _CHUNK, sz * _CHUNK)],
                wsem,
            )
        for d in wdescs:
            if d is not None:
                d.wait()

    return gather_kernel(table2, idx1d)


def _tc_select_matmul(x3, par, w3, b, bt=4096):
    nd, bsz, pack = x3.shape
    n = w3.shape[2]

    def body(x_ref, p_ref, w_ref, b_ref, o_ref):
        d = pl.program_id(1)
        xb = x_ref[0]
        dmask = lax.broadcasted_iota(jnp.int32, (1, nd), 1) == d
        pf = jnp.sum(p_ref[...] * dmask.astype(jnp.float32), axis=1, keepdims=True)
        sel = jnp.where(pf == 1.0, xb[:, pack // 2 :], xb[:, : pack // 2])
        acc = lax.dot_general(
            sel.astype(jnp.bfloat16), w_ref[0], (((1,), (0,)), ((), ())),
            preferred_element_type=jnp.float32,
        )

        @pl.when(d == 0)
        def _():
            o_ref[...] = b_ref[...] + acc

        @pl.when(d != 0)
        def _():
            o_ref[...] += acc

    return pl.pallas_call(
        body,
        grid=(bsz // bt, nd),
        in_specs=[
            pl.BlockSpec((1, bt, pack), lambda i, d: (d, i, 0)),
            pl.BlockSpec((bt, nd), lambda i, d: (i, 0)),
            pl.BlockSpec((1, pack // 2, n), lambda i, d: (d, 0, 0)),
            pl.BlockSpec((1, n), lambda i, d: (0, 0)),
        ],
        out_specs=pl.BlockSpec((bt, n), lambda i, d: (i, 0)),
        out_shape=jax.ShapeDtypeStruct((bsz, n), jnp.float32),
    )(x3, par, w3, b)


def kernel(input_x, emb_table, W_fc, b_fc):
    bsz, nd = input_x.shape
    vocab, emb = emb_table.shape
    out_dim = W_fc.shape[0]
    total = bsz * nd

    table2 = emb_table.reshape(vocab // 2, 2 * emb)
    idx_t = input_x.T.reshape(-1)
    gathered = _sc_gather_packed(table2, idx_t >> 1, total)
    x3 = gathered.reshape(nd, bsz, 2 * emb)
    par = (input_x & 1).astype(jnp.float32)
    w3 = W_fc.T.reshape(nd, emb, out_dim).astype(jnp.bfloat16)
    return _tc_select_matmul(x3, par, w3, b_fc.reshape(1, out_dim))


# speedup vs baseline: 1.2915x; 1.0208x over previous
import functools

import jax
import jax.numpy as jnp
from jax import lax
from jax.experimental import pallas as pl
from jax.experimental.pallas import tpu as pltpu
from jax.experimental.pallas import tpu_sc as plsc

_NUM_WORKERS = 32
_CHUNK = 128
_RCH = 3


_NBUF = 4


def _sc_gather_packed(table2, idx_prep, bsz, nd):
    pack = table2.shape[1]
    bpw = bsz // _NUM_WORKERS
    mesh = plsc.VectorSubcoreMesh(core_axis_name="c", subcore_axis_name="s")

    @functools.partial(
        pl.kernel,
        mesh=mesh,
        out_type=jax.ShapeDtypeStruct((nd * bsz, pack), jnp.float32),
        compiler_params=pltpu.CompilerParams(use_tc_tiling_on_sc=False),
        scratch_types=[
            pltpu.VMEM((nd * bpw,), jnp.int32),
        ]
        + [pltpu.VMEM((_CHUNK, pack), jnp.float32) for _ in range(_NBUF)]
        + [pltpu.SemaphoreType.DMA for _ in range(2 * _NBUF)],
    )
    def gather_kernel(table_hbm, idx_hbm, out_hbm, idxt_v, *rest):
        bufs = rest[:_NBUF]
        gsems = rest[_NBUF : 2 * _NBUF]
        wsems = rest[2 * _NBUF : 3 * _NBUF]
        wid = lax.axis_index("s") * 2 + lax.axis_index("c")
        b0 = wid * bpw
        pltpu.sync_copy(idx_hbm.at[pl.ds(wid * nd * bpw, nd * bpw)], idxt_v)

        gdescs = [None] * nd
        wdescs = [None] * nd
        for d in range(nd):
            s = d % _NBUF
            if d >= _NBUF:
                wdescs[d - _NBUF].wait()
            gdescs[d] = pltpu.async_copy(
                table_hbm.at[idxt_v.at[pl.ds(d * bpw, bpw)]],
                bufs[s],
                gsems[s],
            )
            if d >= 1:
                gdescs[d - 1].wait()
                wdescs[d - 1] = pltpu.async_copy(
                    bufs[(d - 1) % _NBUF],
                    out_hbm.at[pl.ds((d - 1) * bsz + b0, bpw)],
                    wsems[(d - 1) % _NBUF],
                )
        gdescs[nd - 1].wait()
        wdescs[nd - 1] = pltpu.async_copy(
            bufs[(nd - 1) % _NBUF],
            out_hbm.at[pl.ds((nd - 1) * bsz + b0, bpw)],
            wsems[(nd - 1) % _NBUF],
        )
        for d in range(nd - _NBUF, nd):
            wdescs[d].wait()

    return gather_kernel(table2, idx_prep)


def _tc_select_matmul(x3, par, w3, b, bt=4096):
    nd, bsz, pack = x3.shape
    n = w3.shape[2]

    def body(x_ref, p_ref, w_ref, b_ref, o_ref):
        d = pl.program_id(1)
        xb = x_ref[0]
        dmask = lax.broadcasted_iota(jnp.int32, (1, nd), 1) == d
        pf = jnp.sum(p_ref[...] * dmask.astype(jnp.float32), axis=1, keepdims=True)
        sel = jnp.where(pf == 1.0, xb[:, pack // 2 :], xb[:, : pack // 2])
        acc = lax.dot_general(
            sel.astype(jnp.bfloat16), w_ref[0], (((1,), (0,)), ((), ())),
            preferred_element_type=jnp.float32,
        )

        @pl.when(d == 0)
        def _():
            o_ref[...] = b_ref[...] + acc

        @pl.when(d != 0)
        def _():
            o_ref[...] += acc

    return pl.pallas_call(
        body,
        grid=(bsz // bt, nd),
        in_specs=[
            pl.BlockSpec((1, bt, pack), lambda i, d: (d, i, 0)),
            pl.BlockSpec((bt, nd), lambda i, d: (i, 0)),
            pl.BlockSpec((1, pack // 2, n), lambda i, d: (d, 0, 0)),
            pl.BlockSpec((1, n), lambda i, d: (0, 0)),
        ],
        out_specs=pl.BlockSpec((bt, n), lambda i, d: (i, 0)),
        out_shape=jax.ShapeDtypeStruct((bsz, n), jnp.float32),
    )(x3, par, w3, b)


def kernel(input_x, emb_table, W_fc, b_fc):
    bsz, nd = input_x.shape
    vocab, emb = emb_table.shape
    out_dim = W_fc.shape[0]
    total = bsz * nd

    table2 = emb_table.reshape(vocab // 2, 2 * emb)
    bpw = bsz // _NUM_WORKERS
    idx_prep = (
        (input_x >> 1)
        .T.reshape(nd, _NUM_WORKERS, bpw)
        .transpose(1, 0, 2)
        .reshape(total)
    )
    gathered = _sc_gather_packed(table2, idx_prep, bsz, nd)
    x3 = gathered.reshape(nd, bsz, 2 * emb)
    par = (input_x & 1).astype(jnp.float32)
    w3 = W_fc.T.reshape(nd, emb, out_dim).astype(jnp.bfloat16)
    return _tc_select_matmul(x3, par, w3, b_fc.reshape(1, out_dim))
